# baseline (speedup 1.0000x reference)
import jax
import jax.numpy as jnp
from jax import lax
from jax.experimental import pallas as pl
from jax.experimental.pallas import tpu as pltpu
from jax.experimental.pallas import tpu_sc as plsc

_N_ROWS = 16384
_N_DIMS = 1000
_NC = 2
_NS = 16
_NW = _NC * _NS
_ROWS_PER_W = _N_ROWS // _NW
_RB = 128
_NB = _ROWS_PER_W // _RB
_CS = 200
_NSLICE = _N_DIMS // _CS
_NQ = _NB * _NSLICE
_LANES = 16
_NG = _RB // _LANES
_CU = 8

_NEG_INF = float("-inf")


def _onehot_sc(
    xt_hbm, out_hbm, in0, in1, ot0, ot1, accv, acci, si0, si1, so0, so1
):
    wid = lax.axis_index("s") * _NC + lax.axis_index("c")
    rbase = wid * _ROWS_PER_W
    inb = (in0, in1)
    otb = (ot0, ot1)
    sin = (si0, si1)
    sout = (so0, so1)

    def in_copy(q, i):
        b = q // _NSLICE
        s = q - b * _NSLICE
        return pltpu.make_async_copy(
            xt_hbm.at[pl.ds(s * _CS, _CS), pl.ds(rbase + b * _RB, _RB)],
            inb[i],
            sin[i],
        )

    def out_copy(b, s2, p):
        return pltpu.make_async_copy(
            otb[p],
            out_hbm.at[pl.ds(s2 * _CS, _CS), pl.ds(rbase + b * _RB, _RB)],
            sout[p],
        )

    in_copy(0, 0).start()
    in_copy(1, 1).start()

    neg_inf = jnp.full((_LANES,), _NEG_INF, dtype=jnp.float32)
    zero_i = jnp.zeros((_LANES,), dtype=jnp.int32)
    one = jnp.full((_LANES,), 1.0, dtype=jnp.float32)
    zero = jnp.zeros((_LANES,), dtype=jnp.float32)

    def bb_body(bb, carry):
        for b2 in range(2):
            b = bb * 2 + b2
            q0 = b * _NSLICE

            for s in range(_NSLICE):
                i = (b2 + s) % 2
                po = (b2 + s + 1) % 2
                q = q0 + s
                in_copy(q, i).wait()

                @pl.when(q >= 7)
                def _wait_out():
                    out_copy(b - 1, s, po).wait()

                ref = inb[i]
                oref = otb[po]
                cc0 = jnp.full((_LANES,), s * _CS, dtype=jnp.int32)

                if s == 0:
                    acc0 = (neg_inf,) * _NG + (zero_i,) * _NG
                else:
                    acc0 = tuple(
                        accv[g, pl.ds(0, _LANES)] for g in range(_NG)
                    ) + tuple(
                        acci[b2, g, pl.ds(0, _LANES)] for g in range(_NG)
                    )

                def run(emit, acc0=acc0, cc0=cc0, ref=ref, oref=oref):
                    def c_body(ci, st):
                        avs = list(st[:_NG])
                        ais = list(st[_NG: 2 * _NG])
                        cc = st[2 * _NG]
                        c0 = pl.multiple_of(ci * _CU, _CU)
                        aip = [
                            acci[1 - b2, g, pl.ds(0, _LANES)]
                            for g in range(_NG)
                        ]
                        for u in range(_CU):
                            c = c0 + u
                            for g in range(_NG):
                                v = ref[c, pl.ds(16 * g, _LANES)]
                                upd = v > avs[g]
                                avs[g] = jnp.maximum(avs[g], v)
                                ais[g] = jnp.where(upd, cc, ais[g])
                            if emit:
                                for g in range(_NG):
                                    hit = aip[g] == cc
                                    oref[c, pl.ds(16 * g, _LANES)] = (
                                        jnp.where(hit, one, zero)
                                    )
                            cc = cc + 1
                        return tuple(avs) + tuple(ais) + (cc,)

                    st = lax.fori_loop(0, _CS // _CU, c_body, acc0 + (cc0,))
                    for g in range(_NG):
                        accv[g, pl.ds(0, _LANES)] = st[g]
                        acci[b2, g, pl.ds(0, _LANES)] = st[_NG + g]

                if b2 == 1:
                    run(True)
                else:
                    @pl.when(bb > 0)
                    def _fused():
                        run(True)

                    @pl.when(bb == 0)
                    def _scan_only():
                        run(False)

                @pl.when(b > 0)
                def _start_out():
                    out_copy(b - 1, s, po).start()

                @pl.when(q + 2 < _NQ)
                def _next_in():
                    in_copy(q + 2, i).start()
        return carry

    lax.fori_loop(0, _NB // 2, bb_body, 0)

    ai = [acci[1, g, pl.ds(0, _LANES)] for g in range(_NG)]
    for s2 in range(_NSLICE):
        p = (_NB - 1 + s2) % 2
        out_copy(_NB - 2, s2, p).wait()

        oref = otb[p]
        base2 = s2 * _CS

        def o_body(ci, carry2, oref=oref, base2=base2):
            c0 = pl.multiple_of(ci * _CU, _CU)
            for u in range(_CU):
                c = c0 + u
                cc = jnp.full((_LANES,), base2 + c, dtype=jnp.int32)
                for g in range(_NG):
                    hit = ai[g] == cc
                    oref[c, pl.ds(16 * g, _LANES)] = jnp.where(hit, one, zero)
            return carry2

        lax.fori_loop(0, _CS // _CU, o_body, 0)
        out_copy(_NB - 1, s2, p).start()

    for s2 in range(_NSLICE - 2, _NSLICE):
        out_copy(_NB - 1, s2, (_NB - 1 + s2) % 2).wait()


@jax.jit
def _onehot_t(xt):
    mesh = plsc.VectorSubcoreMesh(
        core_axis_name="c", subcore_axis_name="s", num_cores=_NC, num_subcores=_NS
    )
    return pl.kernel(
        _onehot_sc,
        out_type=jax.ShapeDtypeStruct((_N_DIMS, _N_ROWS), jnp.float32),
        mesh=mesh,
        scratch_types=[
            pltpu.VMEM((_CS, _RB), jnp.float32),
            pltpu.VMEM((_CS, _RB), jnp.float32),
            pltpu.VMEM((_CS, _RB), jnp.float32),
            pltpu.VMEM((_CS, _RB), jnp.float32),
            pltpu.VMEM((_NG, _LANES), jnp.float32),
            pltpu.VMEM((2, _NG, _LANES), jnp.int32),
            pltpu.SemaphoreType.DMA,
            pltpu.SemaphoreType.DMA,
            pltpu.SemaphoreType.DMA,
            pltpu.SemaphoreType.DMA,
        ],
    )(xt)


def kernel(x):
    out_t = _onehot_t(x.T)
    return out_t.T.reshape(_N_ROWS, 1, _N_DIMS)
